# baseline (speedup 1.0000x reference)
import functools

import jax
import jax.numpy as jnp
from jax.experimental import pallas as pl
from jax.experimental.pallas import tpu as pltpu

NODE_DIM = 512
HID_DIM = 2048
OUT_DIM = 512
N_TOKENS = 32768
NUM_SEGMENTS = 16

BLOCK_T = 1024
NUM_BLOCKS = N_TOKENS // BLOCK_T


def _mlpagg_kernel(seg_ref, x_ref, W1_ref, b1_ref, W2_ref, b2_ref,
                   W3_ref, b3_ref, out_ref, acc_ref, cnt_ref):
    i = pl.program_id(0)

    @pl.when(i == 0)
    def _init():
        acc_ref[...] = jnp.zeros_like(acc_ref)
        cnt_ref[...] = jnp.zeros_like(cnt_ref)

    x = x_ref[...]
    h = jnp.dot(x, W1_ref[...], preferred_element_type=jnp.float32)
    h = jnp.maximum(h + b1_ref[...], 0.0).astype(jnp.bfloat16)
    h = jnp.dot(h, W2_ref[...], preferred_element_type=jnp.float32)
    h = jnp.maximum(h + b2_ref[...], 0.0).astype(jnp.bfloat16)

    seg = seg_ref[0, 0, :]
    onehot = (seg[None, :] == jax.lax.broadcasted_iota(
        jnp.int32, (NUM_SEGMENTS, BLOCK_T), 0)).astype(jnp.bfloat16)
    acc_ref[...] += jnp.dot(onehot, h, preferred_element_type=jnp.float32)
    cnt_ref[...] += jnp.sum(onehot.astype(jnp.float32), axis=1, keepdims=True)

    @pl.when(i == NUM_BLOCKS - 1)
    def _finish():
        counts = jnp.maximum(cnt_ref[:, 0:1], 1.0)
        mean = acc_ref[...] / counts
        out_ref[...] = jnp.dot(
            mean, W3_ref[...], preferred_element_type=jnp.float32) + b3_ref[...]


@jax.jit
def kernel(x, x_batch, W1, b1, W2, b2, W3, b3):
    seg = x_batch.astype(jnp.int32).reshape(NUM_BLOCKS, 1, BLOCK_T)
    x = x.astype(jnp.bfloat16)
    W1 = W1.astype(jnp.bfloat16)
    W2 = W2.astype(jnp.bfloat16)
    b1 = b1.reshape(1, HID_DIM)
    b2 = b2.reshape(1, HID_DIM)
    b3 = b3.reshape(1, OUT_DIM)

    grid = (NUM_BLOCKS,)
    out = pl.pallas_call(
        _mlpagg_kernel,
        grid=grid,
        in_specs=[
            pl.BlockSpec((1, 1, BLOCK_T), lambda i: (i, 0, 0)),
            pl.BlockSpec((BLOCK_T, NODE_DIM), lambda i: (i, 0)),
            pl.BlockSpec((NODE_DIM, HID_DIM), lambda i: (0, 0)),
            pl.BlockSpec((1, HID_DIM), lambda i: (0, 0)),
            pl.BlockSpec((HID_DIM, HID_DIM), lambda i: (0, 0)),
            pl.BlockSpec((1, HID_DIM), lambda i: (0, 0)),
            pl.BlockSpec((HID_DIM, OUT_DIM), lambda i: (0, 0)),
            pl.BlockSpec((1, OUT_DIM), lambda i: (0, 0)),
        ],
        out_specs=pl.BlockSpec((NUM_SEGMENTS, OUT_DIM), lambda i: (0, 0)),
        out_shape=jax.ShapeDtypeStruct((NUM_SEGMENTS, OUT_DIM), jnp.float32),
        scratch_shapes=[
            pltpu.VMEM((NUM_SEGMENTS, HID_DIM), jnp.float32),
            pltpu.VMEM((NUM_SEGMENTS, 128), jnp.float32),
        ],
        compiler_params=pltpu.CompilerParams(
            dimension_semantics=("arbitrary",),
        ),
    )(seg, x, W1, b1, W2, b2, W3, b3)
    return out


# speedup vs baseline: 1.1133x; 1.1133x over previous
import functools

import jax
import jax.numpy as jnp
from jax.experimental import pallas as pl
from jax.experimental.pallas import tpu as pltpu

NODE_DIM = 512
HID_DIM = 2048
OUT_DIM = 512
N_TOKENS = 32768
NUM_SEGMENTS = 16

BLOCK_T = 2048
NUM_BLOCKS = N_TOKENS // BLOCK_T


def _mlpagg_kernel(seg_ref, x_ref, W1_ref, b1_ref, W2_ref, b2_ref,
                   W3_ref, b3_ref, out_ref, acc_ref, cnt_ref):
    i = pl.program_id(0)

    @pl.when(i == 0)
    def _init():
        acc_ref[...] = jnp.zeros_like(acc_ref)
        cnt_ref[...] = jnp.zeros_like(cnt_ref)

    x = x_ref[...]
    h = jnp.dot(x, W1_ref[...], preferred_element_type=jnp.float32)
    h = jnp.maximum(h + b1_ref[...], 0.0)
    h = jnp.dot(h, W2_ref[...], preferred_element_type=jnp.float32)
    h = jnp.maximum(h + b2_ref[...], 0.0)

    seg = seg_ref[0, 0, :]
    onehot = (seg[None, :] == jax.lax.broadcasted_iota(
        jnp.int32, (NUM_SEGMENTS, BLOCK_T), 0)).astype(jnp.float32)
    acc_ref[...] += jnp.dot(onehot, h, preferred_element_type=jnp.float32)
    cnt_ref[...] += jnp.sum(onehot, axis=1, keepdims=True)

    @pl.when(i == NUM_BLOCKS - 1)
    def _finish():
        counts = jnp.maximum(cnt_ref[:, 0:1], 1.0)
        mean = acc_ref[...] / counts
        out_ref[...] = jnp.dot(
            mean, W3_ref[...], preferred_element_type=jnp.float32) + b3_ref[...]


@jax.jit
def kernel(x, x_batch, W1, b1, W2, b2, W3, b3):
    seg = x_batch.astype(jnp.int32).reshape(NUM_BLOCKS, 1, BLOCK_T)
    b1 = b1.reshape(1, HID_DIM)
    b2 = b2.reshape(1, HID_DIM)
    b3 = b3.reshape(1, OUT_DIM)

    grid = (NUM_BLOCKS,)
    out = pl.pallas_call(
        _mlpagg_kernel,
        grid=grid,
        in_specs=[
            pl.BlockSpec((1, 1, BLOCK_T), lambda i: (i, 0, 0)),
            pl.BlockSpec((BLOCK_T, NODE_DIM), lambda i: (i, 0)),
            pl.BlockSpec((NODE_DIM, HID_DIM), lambda i: (0, 0)),
            pl.BlockSpec((1, HID_DIM), lambda i: (0, 0)),
            pl.BlockSpec((HID_DIM, HID_DIM), lambda i: (0, 0)),
            pl.BlockSpec((1, HID_DIM), lambda i: (0, 0)),
            pl.BlockSpec((HID_DIM, OUT_DIM), lambda i: (0, 0)),
            pl.BlockSpec((1, OUT_DIM), lambda i: (0, 0)),
        ],
        out_specs=pl.BlockSpec((NUM_SEGMENTS, OUT_DIM), lambda i: (0, 0)),
        out_shape=jax.ShapeDtypeStruct((NUM_SEGMENTS, OUT_DIM), jnp.float32),
        scratch_shapes=[
            pltpu.VMEM((NUM_SEGMENTS, HID_DIM), jnp.float32),
            pltpu.VMEM((NUM_SEGMENTS, 128), jnp.float32),
        ],
        compiler_params=pltpu.CompilerParams(
            dimension_semantics=("arbitrary",),
        ),
    )(seg, x, W1, b1, W2, b2, W3, b3)
    return out


# speedup vs baseline: 1.1210x; 1.0069x over previous
import functools

import jax
import jax.numpy as jnp
from jax.experimental import pallas as pl
from jax.experimental.pallas import tpu as pltpu

NODE_DIM = 512
HID_DIM = 2048
OUT_DIM = 512
N_TOKENS = 32768
NUM_SEGMENTS = 16

BLOCK_T = 2048
NUM_BLOCKS = N_TOKENS // BLOCK_T
N_CHUNKS = 2
CHUNK_T = BLOCK_T // N_CHUNKS


def _mlpagg_kernel(seg_ref, x_ref, W1_ref, b1_ref, W2_ref, b2_ref,
                   W3_ref, b3_ref, out_ref, acc_ref, cnt_ref):
    i = pl.program_id(0)

    @pl.when(i == 0)
    def _init():
        acc_ref[...] = jnp.zeros_like(acc_ref)
        cnt_ref[...] = jnp.zeros_like(cnt_ref)

    seg = seg_ref[0, 0, :]
    onehot = (seg[None, :] == jax.lax.broadcasted_iota(
        jnp.int32, (NUM_SEGMENTS, BLOCK_T), 0)).astype(jnp.float32)
    cnt_ref[...] += jnp.sum(onehot, axis=1, keepdims=True)

    acc = jnp.zeros((NUM_SEGMENTS, HID_DIM), dtype=jnp.float32)
    for c in range(N_CHUNKS):
        sl = slice(c * CHUNK_T, (c + 1) * CHUNK_T)
        x = x_ref[sl, :]
        h = jnp.dot(x, W1_ref[...], preferred_element_type=jnp.float32)
        h = jnp.maximum(h + b1_ref[...], 0.0)
        h = jnp.dot(h, W2_ref[...], preferred_element_type=jnp.float32)
        h = jnp.maximum(h + b2_ref[...], 0.0)
        acc = acc + jnp.dot(onehot[:, sl], h,
                            preferred_element_type=jnp.float32)
    acc_ref[...] += acc

    @pl.when(i == NUM_BLOCKS - 1)
    def _finish():
        counts = jnp.maximum(cnt_ref[:, 0:1], 1.0)
        mean = acc_ref[...] / counts
        out_ref[...] = jnp.dot(
            mean, W3_ref[...], preferred_element_type=jnp.float32) + b3_ref[...]


@jax.jit
def kernel(x, x_batch, W1, b1, W2, b2, W3, b3):
    seg = x_batch.astype(jnp.int32).reshape(NUM_BLOCKS, 1, BLOCK_T)
    b1 = b1.reshape(1, HID_DIM)
    b2 = b2.reshape(1, HID_DIM)
    b3 = b3.reshape(1, OUT_DIM)

    grid = (NUM_BLOCKS,)
    out = pl.pallas_call(
        _mlpagg_kernel,
        grid=grid,
        in_specs=[
            pl.BlockSpec((1, 1, BLOCK_T), lambda i: (i, 0, 0)),
            pl.BlockSpec((BLOCK_T, NODE_DIM), lambda i: (i, 0)),
            pl.BlockSpec((NODE_DIM, HID_DIM), lambda i: (0, 0)),
            pl.BlockSpec((1, HID_DIM), lambda i: (0, 0)),
            pl.BlockSpec((HID_DIM, HID_DIM), lambda i: (0, 0)),
            pl.BlockSpec((1, HID_DIM), lambda i: (0, 0)),
            pl.BlockSpec((HID_DIM, OUT_DIM), lambda i: (0, 0)),
            pl.BlockSpec((1, OUT_DIM), lambda i: (0, 0)),
        ],
        out_specs=pl.BlockSpec((NUM_SEGMENTS, OUT_DIM), lambda i: (0, 0)),
        out_shape=jax.ShapeDtypeStruct((NUM_SEGMENTS, OUT_DIM), jnp.float32),
        scratch_shapes=[
            pltpu.VMEM((NUM_SEGMENTS, HID_DIM), jnp.float32),
            pltpu.VMEM((NUM_SEGMENTS, 128), jnp.float32),
        ],
        compiler_params=pltpu.CompilerParams(
            dimension_semantics=("arbitrary",),
        ),
    )(seg, x, W1, b1, W2, b2, W3, b3)
    return out
